# baseline (speedup 1.0000x reference)
import functools

import jax
import jax.numpy as jnp
from jax import lax
from jax.experimental import pallas as pl
from jax.experimental.pallas import tpu as pltpu
from jax.experimental.pallas import tpu_sc as plsc

NC, NS = 2, 16
NW = NC * NS
L = 16
TOTAL = 16384 * 200
PER_W = TOTAL // NW
NCHUNK = 4
CHUNK = PER_W // NCHUNK
UNROLL = 8
STEPS = CHUNK // (L * UNROLL)

@functools.cache
def _build_sc_partial_sums():
    mesh = plsc.VectorSubcoreMesh(
        core_axis_name="c", subcore_axis_name="s", num_cores=NC, num_subcores=NS
    )
    return functools.partial(
        pl.kernel,
        out_type=jax.ShapeDtypeStruct((NW, 2 * L), jnp.float32),
        mesh=mesh,
        scratch_types=[
            pltpu.VMEM((CHUNK,), jnp.float32),
            pltpu.VMEM((CHUNK,), jnp.float32),
            pltpu.VMEM((L,), jnp.float32),
            pltpu.VMEM((2 * L,), jnp.float32),
        ],
    )(_sc_partial_sums)


def _sc_partial_sums(p_hbm, t_hbm, v_hbm, out_hbm, p_v, t_v, v_v, part_v):
    wid = lax.axis_index("s") * NC + lax.axis_index("c")
    base = wid * PER_W
    pltpu.sync_copy(v_hbm, v_v)
    v = v_v[...]
    num = jnp.zeros((L,), jnp.float32)
    cnt = jnp.zeros((L,), jnp.float32)
    for c in range(NCHUNK):
        pltpu.sync_copy(p_hbm.at[pl.ds(base + c * CHUNK, CHUNK)], p_v)
        pltpu.sync_copy(t_hbm.at[pl.ds(base + c * CHUNK, CHUNK)], t_v)

        def body(i, carry, p_v=p_v, t_v=t_v, v=v):
            num, cnt = carry
            for k in range(UNROLL):
                off = i * (L * UNROLL) + k * L
                t = t_v[pl.ds(off, L)]
                p = p_v[pl.ds(off, L)]
                m = t > v
                safe = jnp.where(m, t, jnp.inf)
                num = num + jnp.abs((t - p) / safe)
                cnt = cnt + jnp.where(m, 1.0, 0.0)
            return num, cnt

        num, cnt = lax.fori_loop(0, STEPS, body, (num, cnt))
    part_v[pl.ds(0, L)] = num
    part_v[pl.ds(L, L)] = cnt
    pltpu.sync_copy(part_v, out_hbm.at[wid])


def _combine_body(parts_ref, o_ref):
    x = parts_ref[...]
    num = jnp.sum(x[:, :L])
    cnt = jnp.sum(x[:, L:])
    o_ref[...] = jnp.broadcast_to(num / cnt, (1, 1))


def kernel(preds, targets, value):
    p_flat = jnp.reshape(preds, (TOTAL,))
    t_flat = jnp.reshape(targets, (TOTAL,))
    v_vec = jnp.full((L,), jnp.asarray(value, jnp.float32))
    parts = _build_sc_partial_sums()(p_flat, t_flat, v_vec)
    out = pl.pallas_call(
        _combine_body,
        out_shape=jax.ShapeDtypeStruct((1, 1), jnp.float32),
    )(parts)
    return out[0, 0]


# speedup vs baseline: 1.4484x; 1.4484x over previous
import functools

import jax
import jax.numpy as jnp
from jax import lax
from jax.experimental import pallas as pl
from jax.experimental.pallas import tpu as pltpu
from jax.experimental.pallas import tpu_sc as plsc

NC, NS = 2, 16
NW = NC * NS
L = 16
ROWS, COLS = 16384, 200
FULL = (COLS // L) * L
ROWS_W = ROWS // NW
CHUNK_R = 128
NCHUNK = ROWS_W // CHUNK_R


@functools.cache
def _build_sc_partial_sums():
    mesh = plsc.VectorSubcoreMesh(
        core_axis_name="c", subcore_axis_name="s", num_cores=NC, num_subcores=NS
    )
    return functools.partial(
        pl.kernel,
        out_type=jax.ShapeDtypeStruct((NW, 2 * L), jnp.float32),
        mesh=mesh,
        scratch_types=[
            pltpu.VMEM((CHUNK_R, COLS), jnp.float32),
            pltpu.VMEM((CHUNK_R, COLS), jnp.float32),
            pltpu.VMEM((L,), jnp.float32),
            pltpu.VMEM((2 * L,), jnp.float32),
        ],
    )(_sc_partial_sums)


def _sc_partial_sums(p_hbm, t_hbm, v_hbm, out_hbm, p_v, t_v, v_v, part_v):
    wid = lax.axis_index("s") * NC + lax.axis_index("c")
    base = wid * ROWS_W
    pltpu.sync_copy(v_hbm, v_v)
    v = v_v[...]
    num = jnp.zeros((L,), jnp.float32)
    cnt = jnp.zeros((L,), jnp.float32)
    def acc(t, p, num, cnt, tail=False):
        if tail:
            t = jnp.where(lax.iota(jnp.int32, L) >= (L - (COLS - FULL)), t, v)
        m = t > v
        safe = jnp.where(m, t, jnp.inf)
        num = num + jnp.abs((t - p) / safe)
        cnt = cnt + jnp.where(m, 1.0, 0.0)
        return num, cnt

    for c in range(NCHUNK):
        r0 = base + c * CHUNK_R
        pltpu.sync_copy(p_hbm.at[pl.ds(r0, CHUNK_R)], p_v)
        pltpu.sync_copy(t_hbm.at[pl.ds(r0, CHUNK_R)], t_v)

        def rows_body(r, carry, p_v=p_v, t_v=t_v):
            num, cnt = carry
            for j in range(FULL // L):
                t = t_v[r, pl.ds(j * L, L)]
                p = p_v[r, pl.ds(j * L, L)]
                num, cnt = acc(t, p, num, cnt)
            t = t_v[r, pl.ds(COLS - L, L)]
            p = p_v[r, pl.ds(COLS - L, L)]
            num, cnt = acc(t, p, num, cnt, tail=True)
            return num, cnt

        num, cnt = lax.fori_loop(0, CHUNK_R, rows_body, (num, cnt))
    part_v[pl.ds(0, L)] = num
    part_v[pl.ds(L, L)] = cnt
    pltpu.sync_copy(part_v, out_hbm.at[wid])


def _combine_body(parts_ref, o_ref):
    x = parts_ref[...]
    num = jnp.sum(x[:, :L])
    cnt = jnp.sum(x[:, L:])
    o_ref[...] = jnp.broadcast_to(num / cnt, (1, 1))


def kernel(preds, targets, value):
    v_vec = jnp.full((L,), jnp.asarray(value, jnp.float32))
    parts = _build_sc_partial_sums()(preds, targets, v_vec)
    out = pl.pallas_call(
        _combine_body,
        out_shape=jax.ShapeDtypeStruct((1, 1), jnp.float32),
    )(parts)
    return out[0, 0]
